# baseline (speedup 1.0000x reference)
import functools

import jax
import jax.numpy as jnp
from jax import lax
from jax.experimental import pallas as pl
from jax.experimental.pallas import tpu as pltpu
from jax.experimental.pallas import tpu_sc as plsc

BLANK = 0
EXPAND_BEAM = 2.3
NEG_INF = -1e9
K = 4
CH = 512
_SENTINEL = -1e30
_INT_MAX = 2**31 - 1


def _top4(vals, ids):
    tv, ti = [], []
    for t in range(K):
        mv = jnp.max(vals, axis=1, keepdims=True)
        mi = jnp.min(jnp.where(vals == mv, ids, _INT_MAX), axis=1,
                     keepdims=True)
        tv.append(mv)
        ti.append(mi)
        if t < K - 1:
            vals = jnp.where(ids == mi, _SENTINEL, vals)
    return jnp.concatenate(tv, axis=1), jnp.concatenate(ti, axis=1)


def _stage_a_kernel(x_ref, cids_ref, lse_ref, tailv_ref, taili_ref,
                    m_ref, s_ref, cm_ref, *, n_cols, nb, n_gath, cm_pad):
    j = pl.program_id(1)
    R, C = x_ref.shape
    cpb = C // CH

    @pl.when(j == 0)
    def _init():
        m_ref[...] = jnp.full((R, 1), _SENTINEL, jnp.float32)
        s_ref[...] = jnp.zeros((R, 1), jnp.float32)

    def _process(x):
        cmx = jnp.concatenate(
            [jnp.max(x[:, t * CH:(t + 1) * CH], axis=1, keepdims=True)
             for t in range(cpb)], axis=1)
        bm = jnp.max(cmx, axis=1, keepdims=True)
        m_old = m_ref[...]
        m_new = jnp.maximum(m_old, bm)
        s_ref[...] = s_ref[...] * jnp.exp(m_old - m_new) + jnp.sum(
            jnp.exp(x - m_new), axis=1, keepdims=True)
        m_ref[...] = m_new
        for t in range(nb):
            @pl.when(j == t)
            def _store():
                cm_ref[:, t * cpb:(t + 1) * cpb] = cmx

    @pl.when(j < nb - 1)
    def _full():
        _process(x_ref[...])

    @pl.when(j == nb - 1)
    def _last():
        x = x_ref[...]
        valid = n_cols - (nb - 1) * C
        if valid < C:
            lane = jax.lax.broadcasted_iota(jnp.int32, (R, C), 1)
            x = jnp.where(lane < valid, x, _SENTINEL)
        _process(x)
        tail_len = n_cols - n_gath * CH
        if tail_len > 0:
            lo = n_gath * CH - (nb - 1) * C
            tid = n_gath * CH + jax.lax.broadcasted_iota(
                jnp.int32, (R, tail_len), 1)
            tv, ti = _top4(x[:, lo:lo + tail_len], tid)
            tailv_ref[...] = tv
            taili_ref[...] = ti
            if n_gath < cm_pad:
                cm_ref[:, n_gath:n_gath + 1] = jnp.full((R, 1), _SENTINEL,
                                                        jnp.float32)
        else:
            tailv_ref[...] = jnp.full((R, K), _SENTINEL, jnp.float32)
            taili_ref[...] = (
                n_cols + jax.lax.broadcasted_iota(jnp.int32, (R, K), 1))
        wid = jax.lax.broadcasted_iota(jnp.int32, (R, cm_pad), 1)
        _, cids = _top4(cm_ref[...], wid)
        cids_ref[...] = cids
        lse_ref[...] = m_ref[...] + jnp.log(s_ref[...])


@functools.partial(jax.jit, static_argnames=("rows_blk", "cols_blk"))
def _stage_a(logits, rows_blk, cols_blk):
    n_rows, n_cols = logits.shape
    nb = pl.cdiv(n_cols, cols_blk)
    n_gath = n_cols // CH
    cm_pad = nb * (cols_blk // CH)
    grid = (n_rows // rows_blk, nb)
    return pl.pallas_call(
        functools.partial(_stage_a_kernel, n_cols=n_cols, nb=nb,
                          n_gath=n_gath, cm_pad=cm_pad),
        grid=grid,
        in_specs=[pl.BlockSpec((rows_blk, cols_blk), lambda i, j: (i, j))],
        out_specs=[
            pl.BlockSpec((rows_blk, K), lambda i, j: (i, 0)),
            pl.BlockSpec((rows_blk, 1), lambda i, j: (i, 0)),
            pl.BlockSpec((rows_blk, K), lambda i, j: (i, 0)),
            pl.BlockSpec((rows_blk, K), lambda i, j: (i, 0)),
        ],
        out_shape=[
            jax.ShapeDtypeStruct((n_rows, K), jnp.int32),
            jax.ShapeDtypeStruct((n_rows, 1), jnp.float32),
            jax.ShapeDtypeStruct((n_rows, K), jnp.float32),
            jax.ShapeDtypeStruct((n_rows, K), jnp.int32),
        ],
        scratch_shapes=[
            pltpu.VMEM((rows_blk, 1), jnp.float32),
            pltpu.VMEM((rows_blk, 1), jnp.float32),
            pltpu.VMEM((rows_blk, cm_pad), jnp.float32),
        ],
    )(logits)


def _sc_extract_row(buf8, compact, rsub, off):
    def mk(r):
        def _b():
            for s in range(CH // 16):
                compact[pl.ds(off + 16 * s, 16)] = buf8[r, pl.ds(16 * s, 16)]
        return _b
    lax.switch(rsub, [mk(r) for r in range(8)])


def _sc_gather_body(cids_hbm, logits_hbm, out_hbm, idx_v, buf0, buf1,
                    compact, sem0, sem1, *, per_worker):
    info = plsc.get_sparse_core_info()
    wid = lax.axis_index("s") * info.num_cores + lax.axis_index("c")
    base = wid * per_worker
    pltpu.sync_copy(cids_hbm.at[pl.ds(base, per_worker)], idx_v)
    lane = lax.iota(jnp.int32, 16)

    def _issue(i, buf, sem):
        vbase = pl.multiple_of((i // 16) * 16, 16)
        vec = idx_v[pl.ds(vbase, 16)]
        cid = lax.reduce_max(jnp.where(lane == i - vbase, vec, 0), axes=(0,))
        start = pl.multiple_of(cid * CH, CH)
        row = (base + i) // K
        tile0 = pl.multiple_of((row // 8) * 8, 8)
        return pltpu.make_async_copy(
            logits_hbm.at[pl.ds(tile0, 8), pl.ds(start, CH)], buf, sem), row

    c0, r0 = _issue(0, buf0, sem0)
    c0.start()

    def _step(h, _):
        i0 = 2 * h
        c_a, row_a = _issue(i0, buf0, sem0)
        c_b, row_b = _issue(i0 + 1, buf1, sem1)
        c_b.start()
        c_a.wait()
        _sc_extract_row(buf0, compact, row_a % 8, i0 * CH)

        @pl.when(i0 + 2 < per_worker)
        def _next():
            c_n, _ = _issue(i0 + 2, buf0, sem0)
            c_n.start()
        c_b.wait()
        _sc_extract_row(buf1, compact, row_b % 8, (i0 + 1) * CH)
        return _

    lax.fori_loop(0, per_worker // 2, _step, 0)
    pltpu.sync_copy(compact, out_hbm.at[pl.ds(base * CH, per_worker * CH)])


def _sc_gather(cids_flat, logits):
    n_gr = cids_flat.shape[0]
    info = plsc.get_sparse_core_info()
    nw = info.num_cores * info.num_subcores
    per_worker = n_gr // nw
    mesh = plsc.VectorSubcoreMesh(core_axis_name="c", subcore_axis_name="s")
    body = functools.partial(_sc_gather_body, per_worker=per_worker)
    return pl.kernel(
        body,
        out_type=jax.ShapeDtypeStruct((n_gr * CH,), jnp.float32),
        mesh=mesh,
        compiler_params=pltpu.CompilerParams(needs_layout_passes=False),
        scratch_types=[
            pltpu.VMEM((per_worker,), jnp.int32),
            pltpu.VMEM((8, CH), jnp.float32),
            pltpu.VMEM((8, CH), jnp.float32),
            pltpu.VMEM((per_worker * CH,), jnp.float32),
            pltpu.SemaphoreType.DMA,
            pltpu.SemaphoreType.DMA,
        ],
    )(cids_flat, logits)


def _stage_c_kernel(g_ref, cids_ref, lse_ref, tailv_ref, taili_ref,
                    vals_ref, idx_ref):
    R, W = g_ref.shape
    starts = cids_ref[...] * CH
    off = jax.lax.broadcasted_iota(jnp.int32, (R, CH), 1)
    gcol = jnp.concatenate(
        [starts[:, t:t + 1] + off for t in range(K)], axis=1)
    tv, ti = _top4(g_ref[...], gcol)
    cv = jnp.concatenate([tv, tailv_ref[...]], axis=1)
    ci = jnp.concatenate([ti, taili_ref[...]], axis=1)
    tv, ti = _top4(cv, ci)
    vv = tv - lse_ref[...]
    is_blank = ti[:, 0:1] == BLANK
    best = jnp.where(is_blank, vv[:, 1:2], vv[:, 0:1])
    keep = vv >= best - EXPAND_BEAM
    vals_ref[...] = jnp.where(keep, vv, NEG_INF)
    idx_ref[...] = ti


@functools.partial(jax.jit, static_argnames=("rows_blk",))
def _stage_c(gathered, cids, lse, tailv, taili, rows_blk):
    n_rows = gathered.shape[0]
    grid = (n_rows // rows_blk,)
    out = pl.pallas_call(
        _stage_c_kernel,
        grid=grid,
        in_specs=[
            pl.BlockSpec((rows_blk, K * CH), lambda i: (i, 0)),
            pl.BlockSpec((rows_blk, K), lambda i: (i, 0)),
            pl.BlockSpec((rows_blk, 1), lambda i: (i, 0)),
            pl.BlockSpec((rows_blk, K), lambda i: (i, 0)),
            pl.BlockSpec((rows_blk, K), lambda i: (i, 0)),
        ],
        out_specs=[
            pl.BlockSpec((rows_blk, K), lambda i: (i, 0)),
            pl.BlockSpec((rows_blk, K), lambda i: (i, 0)),
        ],
        out_shape=[
            jax.ShapeDtypeStruct((n_rows, K), jnp.float32),
            jax.ShapeDtypeStruct((n_rows, K), jnp.int32),
        ],
    )(gathered, cids, lse, tailv, taili)
    return out[0], out[1]


def kernel(logits, k):
    del k
    n_rows, n_cols = logits.shape
    rows_blk_a = 8 if n_rows % 8 == 0 else n_rows
    cids, lse, tailv, taili = _stage_a(logits, rows_blk_a, n_cols)
    gathered = _sc_gather(cids.reshape(-1), logits)
    rows_blk_c = 256 if n_rows % 256 == 0 else n_rows
    return _stage_c(gathered.reshape(n_rows, K * CH), cids, lse, tailv,
                    taili, rows_blk_c)


# speedup vs baseline: 1.1438x; 1.1438x over previous
import functools

import jax
import jax.numpy as jnp
from jax import lax
from jax.experimental import pallas as pl
from jax.experimental.pallas import tpu as pltpu
from jax.experimental.pallas import tpu_sc as plsc

BLANK = 0
EXPAND_BEAM = 2.3
NEG_INF = -1e9
K = 4
CH = 512
_SENTINEL = -1e30
_INT_MAX = 2**31 - 1


def _top4(vals, ids):
    tv, ti = [], []
    for t in range(K):
        mv = jnp.max(vals, axis=1, keepdims=True)
        mi = jnp.min(jnp.where(vals == mv, ids, _INT_MAX), axis=1,
                     keepdims=True)
        tv.append(mv)
        ti.append(mi)
        if t < K - 1:
            vals = jnp.where(ids == mi, _SENTINEL, vals)
    return jnp.concatenate(tv, axis=1), jnp.concatenate(ti, axis=1)


def _stage_a_kernel(x_ref, cids_ref, lse_ref, tailv_ref, taili_ref,
                    m_ref, s_ref, cm_ref, *, n_cols, nb, n_gath, cm_pad):
    j = pl.program_id(1)
    R, C = x_ref.shape
    cpb = C // CH

    @pl.when(j == 0)
    def _init():
        m_ref[...] = jnp.full((R, 1), _SENTINEL, jnp.float32)
        s_ref[...] = jnp.zeros((R, 1), jnp.float32)

    def _process(x):
        cmx = jnp.concatenate(
            [jnp.max(x[:, t * CH:(t + 1) * CH], axis=1, keepdims=True)
             for t in range(cpb)], axis=1)
        bm = jnp.max(cmx, axis=1, keepdims=True)
        m_old = m_ref[...]
        m_new = jnp.maximum(m_old, bm)
        s_ref[...] = s_ref[...] * jnp.exp(m_old - m_new) + jnp.sum(
            jnp.exp(x - m_new), axis=1, keepdims=True)
        m_ref[...] = m_new
        for t in range(nb):
            @pl.when(j == t)
            def _store():
                cm_ref[:, t * cpb:(t + 1) * cpb] = cmx

    @pl.when(j < nb - 1)
    def _full():
        _process(x_ref[...])

    @pl.when(j == nb - 1)
    def _last():
        x = x_ref[...]
        valid = n_cols - (nb - 1) * C
        if valid < C:
            lane = jax.lax.broadcasted_iota(jnp.int32, (R, C), 1)
            x = jnp.where(lane < valid, x, _SENTINEL)
        _process(x)
        tail_len = n_cols - n_gath * CH
        if tail_len > 0:
            lo = n_gath * CH - (nb - 1) * C
            tid = n_gath * CH + jax.lax.broadcasted_iota(
                jnp.int32, (R, tail_len), 1)
            tv, ti = _top4(x[:, lo:lo + tail_len], tid)
            tailv_ref[...] = tv
            taili_ref[...] = ti
            if n_gath < cm_pad:
                cm_ref[:, n_gath:n_gath + 1] = jnp.full((R, 1), _SENTINEL,
                                                        jnp.float32)
        else:
            tailv_ref[...] = jnp.full((R, K), _SENTINEL, jnp.float32)
            taili_ref[...] = (
                n_cols + jax.lax.broadcasted_iota(jnp.int32, (R, K), 1))
        wid = jax.lax.broadcasted_iota(jnp.int32, (R, cm_pad), 1)
        _, cids = _top4(cm_ref[...], wid)
        cids_ref[...] = cids
        lse_ref[...] = m_ref[...] + jnp.log(s_ref[...])


@functools.partial(jax.jit, static_argnames=("rows_blk", "cols_blk"))
def _stage_a(logits, rows_blk, cols_blk):
    n_rows, n_cols = logits.shape
    nb = pl.cdiv(n_cols, cols_blk)
    n_gath = n_cols // CH
    cm_pad = nb * (cols_blk // CH)
    grid = (n_rows // rows_blk, nb)
    return pl.pallas_call(
        functools.partial(_stage_a_kernel, n_cols=n_cols, nb=nb,
                          n_gath=n_gath, cm_pad=cm_pad),
        grid=grid,
        in_specs=[pl.BlockSpec((rows_blk, cols_blk), lambda i, j: (i, j))],
        out_specs=[
            pl.BlockSpec((rows_blk, K), lambda i, j: (i, 0)),
            pl.BlockSpec((rows_blk, 1), lambda i, j: (i, 0)),
            pl.BlockSpec((rows_blk, K), lambda i, j: (i, 0)),
            pl.BlockSpec((rows_blk, K), lambda i, j: (i, 0)),
        ],
        out_shape=[
            jax.ShapeDtypeStruct((n_rows, K), jnp.int32),
            jax.ShapeDtypeStruct((n_rows, 1), jnp.float32),
            jax.ShapeDtypeStruct((n_rows, K), jnp.float32),
            jax.ShapeDtypeStruct((n_rows, K), jnp.int32),
        ],
        scratch_shapes=[
            pltpu.VMEM((rows_blk, 1), jnp.float32),
            pltpu.VMEM((rows_blk, 1), jnp.float32),
            pltpu.VMEM((rows_blk, cm_pad), jnp.float32),
        ],
    )(logits)


def _sc_extract_row(buf8, compact, rsub, off):
    def mk(r):
        def _b():
            for s in range(CH // 16):
                compact[pl.ds(off + 16 * s, 16)] = buf8[r, pl.ds(16 * s, 16)]
        return _b
    lax.switch(rsub, [mk(r) for r in range(8)])


def _sc_gather_body(cids_hbm, logits_hbm, out_hbm, idx_v, buf0, buf1,
                    compact, sem0, sem1, *, per_worker):
    info = plsc.get_sparse_core_info()
    wid = lax.axis_index("s") * info.num_cores + lax.axis_index("c")
    base = wid * per_worker
    pltpu.sync_copy(cids_hbm.at[pl.ds(base, per_worker)], idx_v)
    lane = lax.iota(jnp.int32, 16)

    def _issue(i, buf, sem):
        vbase = pl.multiple_of((i // 16) * 16, 16)
        vec = idx_v[pl.ds(vbase, 16)]
        cid = lax.reduce_max(jnp.where(lane == i - vbase, vec, 0), axes=(0,))
        start = pl.multiple_of(cid * CH, CH)
        row = (base + i) // K
        tile0 = pl.multiple_of((row // 8) * 8, 8)
        return pltpu.make_async_copy(
            logits_hbm.at[pl.ds(tile0, 8), pl.ds(start, CH)], buf, sem), row

    c0, r0 = _issue(0, buf0, sem0)
    c0.start()

    def _step(h, _):
        i0 = 2 * h
        c_a, row_a = _issue(i0, buf0, sem0)
        c_b, row_b = _issue(i0 + 1, buf1, sem1)
        c_b.start()
        c_a.wait()
        _sc_extract_row(buf0, compact, row_a % 8, i0 * CH)

        @pl.when(i0 + 2 < per_worker)
        def _next():
            c_n, _ = _issue(i0 + 2, buf0, sem0)
            c_n.start()
        c_b.wait()
        _sc_extract_row(buf1, compact, row_b % 8, (i0 + 1) * CH)
        return _

    lax.fori_loop(0, per_worker // 2, _step, 0)
    pltpu.sync_copy(compact, out_hbm.at[pl.ds(base * CH, per_worker * CH)])


def _sc_gather(cids_flat, logits):
    n_gr = cids_flat.shape[0]
    info = plsc.get_sparse_core_info()
    nw = info.num_cores * info.num_subcores
    per_worker = n_gr // nw
    mesh = plsc.VectorSubcoreMesh(core_axis_name="c", subcore_axis_name="s")
    body = functools.partial(_sc_gather_body, per_worker=per_worker)
    return pl.kernel(
        body,
        out_type=jax.ShapeDtypeStruct((n_gr * CH,), jnp.float32),
        mesh=mesh,
        compiler_params=pltpu.CompilerParams(needs_layout_passes=False),
        scratch_types=[
            pltpu.VMEM((per_worker,), jnp.int32),
            pltpu.VMEM((8, CH), jnp.float32),
            pltpu.VMEM((8, CH), jnp.float32),
            pltpu.VMEM((per_worker * CH,), jnp.float32),
            pltpu.SemaphoreType.DMA,
            pltpu.SemaphoreType.DMA,
        ],
    )(cids_flat, logits)


def _stage_c_kernel(g_ref, cids_ref, lse_ref, tailv_ref, taili_ref,
                    vals_ref, idx_ref):
    R, W = g_ref.shape
    starts = cids_ref[...] * CH
    off = jax.lax.broadcasted_iota(jnp.int32, (R, CH), 1)
    gcol = jnp.concatenate(
        [starts[:, t:t + 1] + off for t in range(K)], axis=1)
    tv, ti = _top4(g_ref[...], gcol)
    cv = jnp.concatenate([tv, tailv_ref[...]], axis=1)
    ci = jnp.concatenate([ti, taili_ref[...]], axis=1)
    tv, ti = _top4(cv, ci)
    vv = tv - lse_ref[...]
    is_blank = ti[:, 0:1] == BLANK
    best = jnp.where(is_blank, vv[:, 1:2], vv[:, 0:1])
    keep = vv >= best - EXPAND_BEAM
    vals_ref[...] = jnp.where(keep, vv, NEG_INF)
    idx_ref[...] = ti


@functools.partial(jax.jit, static_argnames=("rows_blk",))
def _stage_c(gathered, cids, lse, tailv, taili, rows_blk):
    n_rows = gathered.shape[0]
    grid = (n_rows // rows_blk,)
    out = pl.pallas_call(
        _stage_c_kernel,
        grid=grid,
        in_specs=[
            pl.BlockSpec((rows_blk, K * CH), lambda i: (i, 0)),
            pl.BlockSpec((rows_blk, K), lambda i: (i, 0)),
            pl.BlockSpec((rows_blk, 1), lambda i: (i, 0)),
            pl.BlockSpec((rows_blk, K), lambda i: (i, 0)),
            pl.BlockSpec((rows_blk, K), lambda i: (i, 0)),
        ],
        out_specs=[
            pl.BlockSpec((rows_blk, K), lambda i: (i, 0)),
            pl.BlockSpec((rows_blk, K), lambda i: (i, 0)),
        ],
        out_shape=[
            jax.ShapeDtypeStruct((n_rows, K), jnp.float32),
            jax.ShapeDtypeStruct((n_rows, K), jnp.int32),
        ],
    )(gathered, cids, lse, tailv, taili)
    return out[0], out[1]


def kernel(logits, k):
    del k
    n_rows, n_cols = logits.shape
    rows_blk_a = 512 if n_rows % 512 == 0 else n_rows
    cols_blk = 4096 if n_cols > 4096 else n_cols
    cids, lse, tailv, taili = _stage_a(logits, rows_blk_a, cols_blk)
    gathered = _sc_gather(cids.reshape(-1), logits)
    rows_blk_c = 256 if n_rows % 256 == 0 else n_rows
    return _stage_c(gathered.reshape(n_rows, K * CH), cids, lse, tailv,
                    taili, rows_blk_c)


# speedup vs baseline: 1.1751x; 1.0273x over previous
import functools

import jax
import jax.numpy as jnp
from jax import lax
from jax.experimental import pallas as pl
from jax.experimental.pallas import tpu as pltpu
from jax.experimental.pallas import tpu_sc as plsc

BLANK = 0
EXPAND_BEAM = 2.3
NEG_INF = -1e9
K = 4
CH = 512
_SENTINEL = -1e30
_INT_MAX = 2**31 - 1


def _top4(vals, ids):
    tv, ti = [], []
    for t in range(K):
        mv = jnp.max(vals, axis=1, keepdims=True)
        mi = jnp.min(jnp.where(vals == mv, ids, _INT_MAX), axis=1,
                     keepdims=True)
        tv.append(mv)
        ti.append(mi)
        if t < K - 1:
            vals = jnp.where(ids == mi, _SENTINEL, vals)
    return jnp.concatenate(tv, axis=1), jnp.concatenate(ti, axis=1)


def _stage_a_kernel(x_ref, cids_ref, lse_ref, tailv_ref, taili_ref,
                    m_ref, s_ref, cm_ref, *, n_cols, nb, n_gath, cm_pad):
    j = pl.program_id(1)
    R, C = x_ref.shape
    cpb = C // CH

    @pl.when(j == 0)
    def _init():
        m_ref[...] = jnp.full((R, 1), _SENTINEL, jnp.float32)
        s_ref[...] = jnp.zeros((R, 1), jnp.float32)

    def _process(x):
        cmx = jnp.concatenate(
            [jnp.max(x[:, t * CH:(t + 1) * CH], axis=1, keepdims=True)
             for t in range(cpb)], axis=1)
        bm = jnp.max(cmx, axis=1, keepdims=True)
        m_old = m_ref[...]
        m_new = jnp.maximum(m_old, bm)
        s_ref[...] = s_ref[...] * jnp.exp(m_old - m_new) + jnp.sum(
            jnp.exp(x - m_new), axis=1, keepdims=True)
        m_ref[...] = m_new
        for t in range(nb):
            @pl.when(j == t)
            def _store():
                cm_ref[:, t * cpb:(t + 1) * cpb] = cmx

    @pl.when(j < nb - 1)
    def _full():
        _process(x_ref[...])

    @pl.when(j == nb - 1)
    def _last():
        x = x_ref[...]
        valid = n_cols - (nb - 1) * C
        if valid < C:
            lane = jax.lax.broadcasted_iota(jnp.int32, (R, C), 1)
            x = jnp.where(lane < valid, x, _SENTINEL)
        _process(x)
        tail_len = n_cols - n_gath * CH
        if tail_len > 0:
            lo = n_gath * CH - (nb - 1) * C
            tid = n_gath * CH + jax.lax.broadcasted_iota(
                jnp.int32, (R, tail_len), 1)
            tv, ti = _top4(x[:, lo:lo + tail_len], tid)
            tailv_ref[...] = tv
            taili_ref[...] = ti
            if n_gath < cm_pad:
                cm_ref[:, n_gath:n_gath + 1] = jnp.full((R, 1), _SENTINEL,
                                                        jnp.float32)
        else:
            tailv_ref[...] = jnp.full((R, K), _SENTINEL, jnp.float32)
            taili_ref[...] = (
                n_cols + jax.lax.broadcasted_iota(jnp.int32, (R, K), 1))
        wid = jax.lax.broadcasted_iota(jnp.int32, (R, cm_pad), 1)
        _, cids = _top4(cm_ref[...], wid)
        cids_ref[...] = cids
        lse_ref[...] = m_ref[...] + jnp.log(s_ref[...])


@functools.partial(jax.jit, static_argnames=("rows_blk", "cols_blk"))
def _stage_a(logits, rows_blk, cols_blk):
    n_rows, n_cols = logits.shape
    nb = pl.cdiv(n_cols, cols_blk)
    n_gath = n_cols // CH
    cm_pad = nb * (cols_blk // CH)
    grid = (n_rows // rows_blk, nb)
    return pl.pallas_call(
        functools.partial(_stage_a_kernel, n_cols=n_cols, nb=nb,
                          n_gath=n_gath, cm_pad=cm_pad),
        grid=grid,
        in_specs=[pl.BlockSpec((rows_blk, cols_blk), lambda i, j: (i, j))],
        out_specs=[
            pl.BlockSpec((rows_blk, K), lambda i, j: (i, 0)),
            pl.BlockSpec((rows_blk, 1), lambda i, j: (i, 0)),
            pl.BlockSpec((rows_blk, K), lambda i, j: (i, 0)),
            pl.BlockSpec((rows_blk, K), lambda i, j: (i, 0)),
        ],
        out_shape=[
            jax.ShapeDtypeStruct((n_rows, K), jnp.int32),
            jax.ShapeDtypeStruct((n_rows, 1), jnp.float32),
            jax.ShapeDtypeStruct((n_rows, K), jnp.float32),
            jax.ShapeDtypeStruct((n_rows, K), jnp.int32),
        ],
        scratch_shapes=[
            pltpu.VMEM((rows_blk, 1), jnp.float32),
            pltpu.VMEM((rows_blk, 1), jnp.float32),
            pltpu.VMEM((rows_blk, cm_pad), jnp.float32),
        ],
    )(logits)


def _sc_extract_row(buf8, compact, rsub, off):
    def mk(r):
        def _b():
            for s in range(CH // 16):
                compact[pl.ds(off + 16 * s, 16)] = buf8[r, pl.ds(16 * s, 16)]
        return _b
    lax.switch(rsub, [mk(r) for r in range(8)])


def _sc_gather_body(cids_hbm, logits_hbm, out_hbm, idx_v, buf0, buf1,
                    compact, sem0, sem1, *, per_worker):
    info = plsc.get_sparse_core_info()
    wid = lax.axis_index("s") * info.num_cores + lax.axis_index("c")
    base = wid * per_worker
    pltpu.sync_copy(cids_hbm.at[pl.ds(base, per_worker)], idx_v)
    lane = lax.iota(jnp.int32, 16)

    def _issue(i, buf, sem):
        vbase = pl.multiple_of((i // 16) * 16, 16)
        vec = idx_v[pl.ds(vbase, 16)]
        cid = lax.reduce_max(jnp.where(lane == i - vbase, vec, 0), axes=(0,))
        start = pl.multiple_of(cid * CH, CH)
        row = (base + i) // K
        tile0 = pl.multiple_of((row // 8) * 8, 8)
        return pltpu.make_async_copy(
            logits_hbm.at[pl.ds(tile0, 8), pl.ds(start, CH)], buf, sem), row

    c0, r0 = _issue(0, buf0, sem0)
    c0.start()

    def _step(h, _):
        i0 = 2 * h
        c_a, row_a = _issue(i0, buf0, sem0)
        c_b, row_b = _issue(i0 + 1, buf1, sem1)
        c_b.start()
        c_a.wait()
        _sc_extract_row(buf0, compact, row_a % 8, i0 * CH)

        @pl.when(i0 + 2 < per_worker)
        def _next():
            c_n, _ = _issue(i0 + 2, buf0, sem0)
            c_n.start()
        c_b.wait()
        _sc_extract_row(buf1, compact, row_b % 8, (i0 + 1) * CH)
        return _

    lax.fori_loop(0, per_worker // 2, _step, 0)
    pltpu.sync_copy(compact, out_hbm.at[pl.ds(base * CH, per_worker * CH)])


def _sc_gather(cids_flat, logits):
    n_gr = cids_flat.shape[0]
    info = plsc.get_sparse_core_info()
    nw = info.num_cores * info.num_subcores
    per_worker = n_gr // nw
    mesh = plsc.VectorSubcoreMesh(core_axis_name="c", subcore_axis_name="s")
    body = functools.partial(_sc_gather_body, per_worker=per_worker)
    return pl.kernel(
        body,
        out_type=jax.ShapeDtypeStruct((n_gr * CH,), jnp.float32),
        mesh=mesh,
        compiler_params=pltpu.CompilerParams(needs_layout_passes=False),
        scratch_types=[
            pltpu.VMEM((per_worker,), jnp.int32),
            pltpu.VMEM((8, CH), jnp.float32),
            pltpu.VMEM((8, CH), jnp.float32),
            pltpu.VMEM((per_worker * CH,), jnp.float32),
            pltpu.SemaphoreType.DMA,
            pltpu.SemaphoreType.DMA,
        ],
    )(cids_flat, logits)


def _stage_c_kernel(g_ref, cids_ref, lse_ref, tailv_ref, taili_ref,
                    vals_ref, idx_ref):
    R, W = g_ref.shape
    starts = cids_ref[...] * CH
    off = jax.lax.broadcasted_iota(jnp.int32, (R, CH), 1)
    gcol = jnp.concatenate(
        [starts[:, t:t + 1] + off for t in range(K)], axis=1)
    tv, ti = _top4(g_ref[...], gcol)
    cv = jnp.concatenate([tv, tailv_ref[...]], axis=1)
    ci = jnp.concatenate([ti, taili_ref[...]], axis=1)
    tv, ti = _top4(cv, ci)
    vv = tv - lse_ref[...]
    is_blank = ti[:, 0:1] == BLANK
    best = jnp.where(is_blank, vv[:, 1:2], vv[:, 0:1])
    keep = vv >= best - EXPAND_BEAM
    vals_ref[...] = jnp.where(keep, vv, NEG_INF)
    idx_ref[...] = ti


@functools.partial(jax.jit, static_argnames=("rows_blk",))
def _stage_c(gathered, cids, lse, tailv, taili, rows_blk):
    n_rows = gathered.shape[0]
    grid = (n_rows // rows_blk,)
    out = pl.pallas_call(
        _stage_c_kernel,
        grid=grid,
        in_specs=[
            pl.BlockSpec((rows_blk, K * CH), lambda i: (i, 0)),
            pl.BlockSpec((rows_blk, K), lambda i: (i, 0)),
            pl.BlockSpec((rows_blk, 1), lambda i: (i, 0)),
            pl.BlockSpec((rows_blk, K), lambda i: (i, 0)),
            pl.BlockSpec((rows_blk, K), lambda i: (i, 0)),
        ],
        out_specs=[
            pl.BlockSpec((rows_blk, K), lambda i: (i, 0)),
            pl.BlockSpec((rows_blk, K), lambda i: (i, 0)),
        ],
        out_shape=[
            jax.ShapeDtypeStruct((n_rows, K), jnp.float32),
            jax.ShapeDtypeStruct((n_rows, K), jnp.int32),
        ],
    )(gathered, cids, lse, tailv, taili)
    return out[0], out[1]


def kernel(logits, k):
    del k
    n_rows, n_cols = logits.shape
    rows_blk_a = n_rows
    cols_blk = 4096 if n_cols > 4096 else n_cols
    cids, lse, tailv, taili = _stage_a(logits, rows_blk_a, cols_blk)
    gathered = _sc_gather(cids.reshape(-1), logits)
    rows_blk_c = 256 if n_rows % 256 == 0 else n_rows
    return _stage_c(gathered.reshape(n_rows, K * CH), cids, lse, tailv,
                    taili, rows_blk_c)


# speedup vs baseline: 1.1810x; 1.0050x over previous
import functools

import jax
import jax.numpy as jnp
from jax import lax
from jax.experimental import pallas as pl
from jax.experimental.pallas import tpu as pltpu
from jax.experimental.pallas import tpu_sc as plsc

BLANK = 0
EXPAND_BEAM = 2.3
NEG_INF = -1e9
K = 4
CH = 512
_SENTINEL = -1e30
_INT_MAX = 2**31 - 1


def _top4(vals, ids):
    tv, ti = [], []
    for t in range(K):
        mv = jnp.max(vals, axis=1, keepdims=True)
        mi = jnp.min(jnp.where(vals == mv, ids, _INT_MAX), axis=1,
                     keepdims=True)
        tv.append(mv)
        ti.append(mi)
        if t < K - 1:
            vals = jnp.where(ids == mi, _SENTINEL, vals)
    return jnp.concatenate(tv, axis=1), jnp.concatenate(ti, axis=1)


def _stage_a_kernel(x_ref, cids_ref, lse_ref, tailv_ref, taili_ref,
                    m_ref, s_ref, cm_ref, *, n_cols, nb, n_gath, cm_pad):
    j = pl.program_id(1)
    R, C = x_ref.shape
    cpb = C // CH

    @pl.when(j == 0)
    def _init():
        m_ref[...] = jnp.full((R, 1), _SENTINEL, jnp.float32)
        s_ref[...] = jnp.zeros((R, 1), jnp.float32)

    def _process(x):
        cmx = jnp.concatenate(
            [jnp.max(x[:, t * CH:(t + 1) * CH], axis=1, keepdims=True)
             for t in range(cpb)], axis=1)
        bm = jnp.max(cmx, axis=1, keepdims=True)
        m_old = m_ref[...]
        m_new = jnp.maximum(m_old, bm)
        s_ref[...] = s_ref[...] * jnp.exp(m_old - m_new) + jnp.sum(
            jnp.exp(x - m_new), axis=1, keepdims=True)
        m_ref[...] = m_new
        for t in range(nb):
            @pl.when(j == t)
            def _store():
                cm_ref[:, t * cpb:(t + 1) * cpb] = cmx

    @pl.when(j < nb - 1)
    def _full():
        _process(x_ref[...])

    @pl.when(j == nb - 1)
    def _last():
        x = x_ref[...]
        valid = n_cols - (nb - 1) * C
        if valid < C:
            lane = jax.lax.broadcasted_iota(jnp.int32, (R, C), 1)
            x = jnp.where(lane < valid, x, _SENTINEL)
        _process(x)
        tail_len = n_cols - n_gath * CH
        if tail_len > 0:
            lo = n_gath * CH - (nb - 1) * C
            tid = n_gath * CH + jax.lax.broadcasted_iota(
                jnp.int32, (R, tail_len), 1)
            tv, ti = _top4(x[:, lo:lo + tail_len], tid)
            tailv_ref[...] = tv
            taili_ref[...] = ti
            if n_gath < cm_pad:
                cm_ref[:, n_gath:n_gath + 1] = jnp.full((R, 1), _SENTINEL,
                                                        jnp.float32)
        else:
            tailv_ref[...] = jnp.full((R, K), _SENTINEL, jnp.float32)
            taili_ref[...] = (
                n_cols + jax.lax.broadcasted_iota(jnp.int32, (R, K), 1))
        wid = jax.lax.broadcasted_iota(jnp.int32, (R, cm_pad), 1)
        _, cids = _top4(cm_ref[...], wid)
        cids_ref[...] = cids
        lse_ref[...] = m_ref[...] + jnp.log(s_ref[...])


@functools.partial(jax.jit, static_argnames=("rows_blk", "cols_blk"))
def _stage_a(logits, rows_blk, cols_blk):
    n_rows, n_cols = logits.shape
    nb = pl.cdiv(n_cols, cols_blk)
    n_gath = n_cols // CH
    cm_pad = nb * (cols_blk // CH)
    grid = (n_rows // rows_blk, nb)
    return pl.pallas_call(
        functools.partial(_stage_a_kernel, n_cols=n_cols, nb=nb,
                          n_gath=n_gath, cm_pad=cm_pad),
        grid=grid,
        in_specs=[pl.BlockSpec((rows_blk, cols_blk), lambda i, j: (i, j))],
        out_specs=[
            pl.BlockSpec((rows_blk, K), lambda i, j: (i, 0)),
            pl.BlockSpec((rows_blk, 1), lambda i, j: (i, 0)),
            pl.BlockSpec((rows_blk, K), lambda i, j: (i, 0)),
            pl.BlockSpec((rows_blk, K), lambda i, j: (i, 0)),
        ],
        out_shape=[
            jax.ShapeDtypeStruct((n_rows, K), jnp.int32),
            jax.ShapeDtypeStruct((n_rows, 1), jnp.float32),
            jax.ShapeDtypeStruct((n_rows, K), jnp.float32),
            jax.ShapeDtypeStruct((n_rows, K), jnp.int32),
        ],
        scratch_shapes=[
            pltpu.VMEM((rows_blk, 1), jnp.float32),
            pltpu.VMEM((rows_blk, 1), jnp.float32),
            pltpu.VMEM((rows_blk, cm_pad), jnp.float32),
        ],
    )(logits)


def _sc_extract_row(buf8, compact, rsub, off):
    def mk(r):
        def _b():
            for s in range(CH // 16):
                compact[pl.ds(off + 16 * s, 16)] = buf8[r, pl.ds(16 * s, 16)]
        return _b
    lax.switch(rsub, [mk(r) for r in range(8)])


def _sc_gather_body(cids_hbm, logits_hbm, out_hbm, idx_v, buf0, buf1,
                    compact, sem0, sem1, *, per_worker):
    info = plsc.get_sparse_core_info()
    wid = lax.axis_index("s") * info.num_cores + lax.axis_index("c")
    base = wid * per_worker
    pltpu.sync_copy(cids_hbm.at[pl.ds(base, per_worker)], idx_v)
    lane = lax.iota(jnp.int32, 16)

    def _issue(i, buf, sem):
        vbase = pl.multiple_of((i // 16) * 16, 16)
        vec = idx_v[pl.ds(vbase, 16)]
        cid = lax.reduce_max(jnp.where(lane == i - vbase, vec, 0), axes=(0,))
        start = pl.multiple_of(cid * CH, CH)
        row = (base + i) // K
        tile0 = pl.multiple_of((row // 8) * 8, 8)
        return pltpu.make_async_copy(
            logits_hbm.at[pl.ds(tile0, 8), pl.ds(start, CH)], buf, sem), row

    c0, r0 = _issue(0, buf0, sem0)
    c0.start()

    def _step(h, _):
        i0 = 2 * h
        c_a, row_a = _issue(i0, buf0, sem0)
        c_b, row_b = _issue(i0 + 1, buf1, sem1)
        c_b.start()
        c_a.wait()
        _sc_extract_row(buf0, compact, row_a % 8, i0 * CH)

        @pl.when(i0 + 2 < per_worker)
        def _next():
            c_n, _ = _issue(i0 + 2, buf0, sem0)
            c_n.start()
        c_b.wait()
        _sc_extract_row(buf1, compact, row_b % 8, (i0 + 1) * CH)
        return _

    lax.fori_loop(0, per_worker // 2, _step, 0)
    pltpu.sync_copy(compact, out_hbm.at[pl.ds(base * CH, per_worker * CH)])


def _sc_gather(cids_flat, logits):
    n_gr = cids_flat.shape[0]
    info = plsc.get_sparse_core_info()
    nw = info.num_cores * info.num_subcores
    per_worker = n_gr // nw
    mesh = plsc.VectorSubcoreMesh(core_axis_name="c", subcore_axis_name="s")
    body = functools.partial(_sc_gather_body, per_worker=per_worker)
    return pl.kernel(
        body,
        out_type=jax.ShapeDtypeStruct((n_gr * CH,), jnp.float32),
        mesh=mesh,
        compiler_params=pltpu.CompilerParams(needs_layout_passes=False),
        scratch_types=[
            pltpu.VMEM((per_worker,), jnp.int32),
            pltpu.VMEM((8, CH), jnp.float32),
            pltpu.VMEM((8, CH), jnp.float32),
            pltpu.VMEM((per_worker * CH,), jnp.float32),
            pltpu.SemaphoreType.DMA,
            pltpu.SemaphoreType.DMA,
        ],
    )(cids_flat, logits)


def _stage_c_kernel(g_ref, cids_ref, lse_ref, tailv_ref, taili_ref,
                    vals_ref, idx_ref):
    R, W = g_ref.shape
    starts = cids_ref[...] * CH
    off = jax.lax.broadcasted_iota(jnp.int32, (R, CH), 1)
    gcol = jnp.concatenate(
        [starts[:, t:t + 1] + off for t in range(K)], axis=1)
    tv, ti = _top4(g_ref[...], gcol)
    cv = jnp.concatenate([tv, tailv_ref[...]], axis=1)
    ci = jnp.concatenate([ti, taili_ref[...]], axis=1)
    tv, ti = _top4(cv, ci)
    vv = tv - lse_ref[...]
    is_blank = ti[:, 0:1] == BLANK
    best = jnp.where(is_blank, vv[:, 1:2], vv[:, 0:1])
    keep = vv >= best - EXPAND_BEAM
    vals_ref[...] = jnp.where(keep, vv, NEG_INF)
    idx_ref[...] = ti


@functools.partial(jax.jit, static_argnames=("rows_blk",))
def _stage_c(gathered, cids, lse, tailv, taili, rows_blk):
    n_rows = gathered.shape[0]
    grid = (n_rows // rows_blk,)
    out = pl.pallas_call(
        _stage_c_kernel,
        grid=grid,
        in_specs=[
            pl.BlockSpec((rows_blk, K * CH), lambda i: (i, 0)),
            pl.BlockSpec((rows_blk, K), lambda i: (i, 0)),
            pl.BlockSpec((rows_blk, 1), lambda i: (i, 0)),
            pl.BlockSpec((rows_blk, K), lambda i: (i, 0)),
            pl.BlockSpec((rows_blk, K), lambda i: (i, 0)),
        ],
        out_specs=[
            pl.BlockSpec((rows_blk, K), lambda i: (i, 0)),
            pl.BlockSpec((rows_blk, K), lambda i: (i, 0)),
        ],
        out_shape=[
            jax.ShapeDtypeStruct((n_rows, K), jnp.float32),
            jax.ShapeDtypeStruct((n_rows, K), jnp.int32),
        ],
    )(gathered, cids, lse, tailv, taili)
    return out[0], out[1]


def kernel(logits, k):
    del k
    n_rows, n_cols = logits.shape
    rows_blk_a = 512 if n_rows % 512 == 0 else n_rows
    cols_blk = 8192 if n_cols > 8192 else n_cols
    cids, lse, tailv, taili = _stage_a(logits, rows_blk_a, cols_blk)
    gathered = _sc_gather(cids.reshape(-1), logits)
    rows_blk_c = 256 if n_rows % 256 == 0 else n_rows
    return _stage_c(gathered.reshape(n_rows, K * CH), cids, lse, tailv,
                    taili, rows_blk_c)


# speedup vs baseline: 1.2312x; 1.0425x over previous
import functools

import jax
import jax.numpy as jnp
from jax import lax
from jax.experimental import pallas as pl
from jax.experimental.pallas import tpu as pltpu
from jax.experimental.pallas import tpu_sc as plsc

BLANK = 0
EXPAND_BEAM = 2.3
NEG_INF = -1e9
K = 4
CH = 512
_SENTINEL = -1e30
_INT_MAX = 2**31 - 1


def _top4(vals, ids):
    tv, ti = [], []
    for t in range(K):
        mv = jnp.max(vals, axis=1, keepdims=True)
        mi = jnp.min(jnp.where(vals == mv, ids, _INT_MAX), axis=1,
                     keepdims=True)
        tv.append(mv)
        ti.append(mi)
        if t < K - 1:
            vals = jnp.where(ids == mi, _SENTINEL, vals)
    return jnp.concatenate(tv, axis=1), jnp.concatenate(ti, axis=1)


def _stage_a_kernel(x_ref, cids_ref, lse_ref, tailv_ref, taili_ref,
                    m_ref, s_ref, cm_ref, *, n_cols, nb, n_gath, cm_pad):
    j = pl.program_id(1)
    R, C = x_ref.shape
    cpb = C // CH

    @pl.when(j == 0)
    def _init():
        m_ref[...] = jnp.full((R, 1), _SENTINEL, jnp.float32)
        s_ref[...] = jnp.zeros((R, 1), jnp.float32)

    def _process(x):
        cmx = jnp.concatenate(
            [jnp.max(x[:, t * CH:(t + 1) * CH], axis=1, keepdims=True)
             for t in range(cpb)], axis=1)
        bm = jnp.max(cmx, axis=1, keepdims=True)
        m_old = m_ref[...]
        m_new = jnp.maximum(m_old, bm)
        s_ref[...] = s_ref[...] * jnp.exp(m_old - m_new) + jnp.sum(
            jnp.exp(x - m_new), axis=1, keepdims=True)
        m_ref[...] = m_new
        for t in range(nb):
            @pl.when(j == t)
            def _store():
                cm_ref[:, t * cpb:(t + 1) * cpb] = cmx

    @pl.when(j < nb - 1)
    def _full():
        _process(x_ref[...])

    @pl.when(j == nb - 1)
    def _last():
        x = x_ref[...]
        valid = n_cols - (nb - 1) * C
        if valid < C:
            lane = jax.lax.broadcasted_iota(jnp.int32, (R, C), 1)
            x = jnp.where(lane < valid, x, _SENTINEL)
        _process(x)
        tail_len = n_cols - n_gath * CH
        if tail_len > 0:
            lo = n_gath * CH - (nb - 1) * C
            tid = n_gath * CH + jax.lax.broadcasted_iota(
                jnp.int32, (R, tail_len), 1)
            tv, ti = _top4(x[:, lo:lo + tail_len], tid)
            tailv_ref[...] = tv
            taili_ref[...] = ti
            if n_gath < cm_pad:
                cm_ref[:, n_gath:n_gath + 1] = jnp.full((R, 1), _SENTINEL,
                                                        jnp.float32)
        else:
            tailv_ref[...] = jnp.full((R, K), _SENTINEL, jnp.float32)
            taili_ref[...] = (
                n_cols + jax.lax.broadcasted_iota(jnp.int32, (R, K), 1))
        wid = jax.lax.broadcasted_iota(jnp.int32, (R, cm_pad), 1)
        _, cids = _top4(cm_ref[...], wid)
        cids_ref[...] = cids
        lse_ref[...] = m_ref[...] + jnp.log(s_ref[...])


@functools.partial(jax.jit,
                   static_argnames=("rows_blk", "cols_blk", "row_off",
                                    "out_rows"))
def _stage_a(logits, rows_blk, cols_blk, row_off=0, out_rows=None):
    n_rows, n_cols = logits.shape
    out_rows = n_rows if out_rows is None else out_rows
    nb = pl.cdiv(n_cols, cols_blk)
    n_gath = n_cols // CH
    cm_pad = nb * (cols_blk // CH)
    grid = (out_rows // rows_blk, nb)
    off_blocks = row_off // rows_blk
    return pl.pallas_call(
        functools.partial(_stage_a_kernel, n_cols=n_cols, nb=nb,
                          n_gath=n_gath, cm_pad=cm_pad),
        grid=grid,
        in_specs=[pl.BlockSpec((rows_blk, cols_blk),
                               lambda i, j: (i + off_blocks, j))],
        out_specs=[
            pl.BlockSpec((rows_blk, K), lambda i, j: (i, 0)),
            pl.BlockSpec((rows_blk, 1), lambda i, j: (i, 0)),
            pl.BlockSpec((rows_blk, K), lambda i, j: (i, 0)),
            pl.BlockSpec((rows_blk, K), lambda i, j: (i, 0)),
        ],
        out_shape=[
            jax.ShapeDtypeStruct((out_rows, K), jnp.int32),
            jax.ShapeDtypeStruct((out_rows, 1), jnp.float32),
            jax.ShapeDtypeStruct((out_rows, K), jnp.float32),
            jax.ShapeDtypeStruct((out_rows, K), jnp.int32),
        ],
        scratch_shapes=[
            pltpu.VMEM((rows_blk, 1), jnp.float32),
            pltpu.VMEM((rows_blk, 1), jnp.float32),
            pltpu.VMEM((rows_blk, cm_pad), jnp.float32),
        ],
    )(logits)


def _sc_extract_row(buf8, compact, rsub, off):
    def mk(r):
        def _b():
            for s in range(CH // 16):
                compact[pl.ds(off + 16 * s, 16)] = buf8[r, pl.ds(16 * s, 16)]
        return _b
    lax.switch(rsub, [mk(r) for r in range(8)])


def _sc_gather_body(cids_hbm, logits_hbm, out_hbm, idx_v, buf0, buf1,
                    compact, sem0, sem1, *, per_worker, row_off):
    info = plsc.get_sparse_core_info()
    wid = lax.axis_index("s") * info.num_cores + lax.axis_index("c")
    base = wid * per_worker
    pltpu.sync_copy(cids_hbm.at[pl.ds(base, per_worker)], idx_v)
    lane = lax.iota(jnp.int32, 16)

    def _issue(i, buf, sem):
        vbase = pl.multiple_of((i // 16) * 16, 16)
        vec = idx_v[pl.ds(vbase, 16)]
        cid = lax.reduce_max(jnp.where(lane == i - vbase, vec, 0), axes=(0,))
        start = pl.multiple_of(cid * CH, CH)
        row = (base + i) // K + row_off
        tile0 = pl.multiple_of((row // 8) * 8, 8)
        return pltpu.make_async_copy(
            logits_hbm.at[pl.ds(tile0, 8), pl.ds(start, CH)], buf, sem), row

    c0, r0 = _issue(0, buf0, sem0)
    c0.start()

    def _step(h, _):
        i0 = 2 * h
        c_a, row_a = _issue(i0, buf0, sem0)
        c_b, row_b = _issue(i0 + 1, buf1, sem1)
        c_b.start()
        c_a.wait()
        _sc_extract_row(buf0, compact, row_a % 8, i0 * CH)

        @pl.when(i0 + 2 < per_worker)
        def _next():
            c_n, _ = _issue(i0 + 2, buf0, sem0)
            c_n.start()
        c_b.wait()
        _sc_extract_row(buf1, compact, row_b % 8, (i0 + 1) * CH)
        return _

    lax.fori_loop(0, per_worker // 2, _step, 0)
    pltpu.sync_copy(compact, out_hbm.at[pl.ds(base * CH, per_worker * CH)])


def _sc_gather(cids_flat, logits, row_off=0):
    n_gr = cids_flat.shape[0]
    info = plsc.get_sparse_core_info()
    nw = info.num_cores * info.num_subcores
    per_worker = n_gr // nw
    mesh = plsc.VectorSubcoreMesh(core_axis_name="c", subcore_axis_name="s")
    body = functools.partial(_sc_gather_body, per_worker=per_worker,
                             row_off=row_off)
    return pl.kernel(
        body,
        out_type=jax.ShapeDtypeStruct((n_gr * CH,), jnp.float32),
        mesh=mesh,
        compiler_params=pltpu.CompilerParams(needs_layout_passes=False),
        scratch_types=[
            pltpu.VMEM((per_worker,), jnp.int32),
            pltpu.VMEM((8, CH), jnp.float32),
            pltpu.VMEM((8, CH), jnp.float32),
            pltpu.VMEM((per_worker * CH,), jnp.float32),
            pltpu.SemaphoreType.DMA,
            pltpu.SemaphoreType.DMA,
        ],
    )(cids_flat, logits)


def _stage_c_kernel(g_ref, cids_ref, lse_ref, tailv_ref, taili_ref,
                    vals_ref, idx_ref):
    R, W = g_ref.shape
    starts = cids_ref[...] * CH
    off = jax.lax.broadcasted_iota(jnp.int32, (R, CH), 1)
    gcol = jnp.concatenate(
        [starts[:, t:t + 1] + off for t in range(K)], axis=1)
    tv, ti = _top4(g_ref[...], gcol)
    cv = jnp.concatenate([tv, tailv_ref[...]], axis=1)
    ci = jnp.concatenate([ti, taili_ref[...]], axis=1)
    tv, ti = _top4(cv, ci)
    vv = tv - lse_ref[...]
    is_blank = ti[:, 0:1] == BLANK
    best = jnp.where(is_blank, vv[:, 1:2], vv[:, 0:1])
    keep = vv >= best - EXPAND_BEAM
    vals_ref[...] = jnp.where(keep, vv, NEG_INF)
    idx_ref[...] = ti


@functools.partial(jax.jit, static_argnames=("rows_blk",))
def _stage_c(gathered, cids, lse, tailv, taili, rows_blk):
    n_rows = gathered.shape[0]
    grid = (n_rows // rows_blk,)
    out = pl.pallas_call(
        _stage_c_kernel,
        grid=grid,
        in_specs=[
            pl.BlockSpec((rows_blk, K * CH), lambda i: (i, 0)),
            pl.BlockSpec((rows_blk, K), lambda i: (i, 0)),
            pl.BlockSpec((rows_blk, 1), lambda i: (i, 0)),
            pl.BlockSpec((rows_blk, K), lambda i: (i, 0)),
            pl.BlockSpec((rows_blk, K), lambda i: (i, 0)),
        ],
        out_specs=[
            pl.BlockSpec((rows_blk, K), lambda i: (i, 0)),
            pl.BlockSpec((rows_blk, K), lambda i: (i, 0)),
        ],
        out_shape=[
            jax.ShapeDtypeStruct((n_rows, K), jnp.float32),
            jax.ShapeDtypeStruct((n_rows, K), jnp.int32),
        ],
    )(gathered, cids, lse, tailv, taili)
    return out[0], out[1]


def kernel(logits, k):
    del k
    n_rows, n_cols = logits.shape
    cols_blk = 8192 if n_cols > 8192 else n_cols
    n_half = n_rows // 2 if n_rows % 1024 == 0 else n_rows
    outs = []
    for h in range(n_rows // n_half):
        off = h * n_half
        rows_blk_a = 512 if n_half % 512 == 0 else n_half
        cids, lse, tailv, taili = _stage_a(
            logits, rows_blk_a, cols_blk, row_off=off, out_rows=n_half)
        gathered = _sc_gather(cids.reshape(-1), logits, row_off=off)
        rows_blk_c = 256 if n_half % 256 == 0 else n_half
        outs.append(_stage_c(gathered.reshape(n_half, K * CH), cids, lse,
                             tailv, taili, rows_blk_c))
    if len(outs) == 1:
        return outs[0]
    return (jnp.concatenate([o[0] for o in outs], axis=0),
            jnp.concatenate([o[1] for o in outs], axis=0))
